# baseline (device time: 79460 ns/iter reference)
import functools

import jax
import jax.numpy as jnp
from jax import lax
from jax.experimental import pallas as pl
from jax.experimental.pallas import tpu as pltpu

N_DEV = 16


def kernel(x, router_W, route_idx, expert_W):
    n, d = x.shape
    e_per, _, h = expert_W.shape

    def body(x_ref, ridx_ref, ew_ref, out_ref, comm_ref, send_sems, recv_sems):
        my = lax.axis_index("i")
        left = lax.rem(my + N_DEV - 1, N_DEV)
        right = lax.rem(my + 1, N_DEV)

        barrier_sem = pltpu.get_barrier_semaphore()
        for nbr in (left, right):
            pl.semaphore_signal(
                barrier_sem, inc=1,
                device_id=(nbr,), device_id_type=pl.DeviceIdType.MESH,
            )
        pl.semaphore_wait(barrier_sem, 2)

        ridx = ridx_ref[:, :1]
        partial = jnp.zeros((n, h), jnp.float32)
        for j in range(e_per):
            e_id = my * e_per + j
            mask = (ridx == e_id).astype(jnp.float32)
            xm = x_ref[:, :] * mask
            partial = partial + jnp.dot(
                xm, ew_ref[j], preferred_element_type=jnp.float32
            )
        out_ref[:, :] = partial
        comm_ref[0] = partial

        for k in range(N_DEV - 1):
            rdma = pltpu.make_async_remote_copy(
                src_ref=comm_ref.at[k],
                dst_ref=comm_ref.at[k + 1],
                send_sem=send_sems.at[k],
                recv_sem=recv_sems.at[k],
                device_id=(right,),
                device_id_type=pl.DeviceIdType.MESH,
            )
            rdma.start()
            rdma.wait()
            out_ref[:, :] += comm_ref[k + 1]

        @functools.partial(
            pl.run_scoped, second_barrier=pltpu.SemaphoreType.REGULAR
        )
        def _(second_barrier):
            for nbr in (left, right):
                pl.semaphore_signal(
                    second_barrier, inc=1,
                    device_id=(nbr,), device_id_type=pl.DeviceIdType.MESH,
                )
            pl.semaphore_wait(second_barrier, 2)

    return pl.pallas_call(
        body,
        out_shape=jax.ShapeDtypeStruct((n, h), jnp.float32),
        in_specs=[
            pl.BlockSpec(memory_space=pltpu.VMEM),
            pl.BlockSpec(memory_space=pltpu.VMEM),
            pl.BlockSpec(memory_space=pltpu.VMEM),
        ],
        out_specs=pl.BlockSpec(memory_space=pltpu.VMEM),
        scratch_shapes=[
            pltpu.VMEM((N_DEV, n, h), jnp.float32),
            pltpu.SemaphoreType.DMA((N_DEV - 1,)),
            pltpu.SemaphoreType.DMA((N_DEV - 1,)),
        ],
        compiler_params=pltpu.CompilerParams(collective_id=0),
    )(x, route_idx, expert_W)


# device time: 30360 ns/iter; 2.6173x vs baseline; 2.6173x over previous
import functools

import jax
import jax.numpy as jnp
from jax import lax
from jax.experimental import pallas as pl
from jax.experimental.pallas import tpu as pltpu

N_DEV = 16
N_ROUNDS = 4


def kernel(x, router_W, route_idx, expert_W):
    n, d = x.shape
    e_per, _, h = expert_W.shape

    def body(x_ref, ridx_ref, ew_ref, out_ref, recv_bufs, send_sems, recv_sems):
        my = lax.axis_index("i")
        partners = [jnp.bitwise_xor(my, 1 << r) for r in range(N_ROUNDS)]

        barrier_sem = pltpu.get_barrier_semaphore()
        for p in partners:
            pl.semaphore_signal(
                barrier_sem, inc=1,
                device_id=(p,), device_id_type=pl.DeviceIdType.MESH,
            )
        pl.semaphore_wait(barrier_sem, N_ROUNDS)

        ridx = ridx_ref[:, :1]
        partial = jnp.zeros((n, h), jnp.float32)
        for j in range(e_per):
            e_id = my * e_per + j
            mask = (ridx == e_id).astype(jnp.float32)
            xm = x_ref[:, :] * mask
            partial = partial + jnp.dot(
                xm, ew_ref[j], preferred_element_type=jnp.float32
            )
        out_ref[:, :] = partial

        for r in range(N_ROUNDS):
            rdma = pltpu.make_async_remote_copy(
                src_ref=out_ref,
                dst_ref=recv_bufs.at[r],
                send_sem=send_sems.at[r],
                recv_sem=recv_sems.at[r],
                device_id=(partners[r],),
                device_id_type=pl.DeviceIdType.MESH,
            )
            rdma.start()
            rdma.wait()
            out_ref[:, :] += recv_bufs[r]

        @functools.partial(
            pl.run_scoped, second_barrier=pltpu.SemaphoreType.REGULAR
        )
        def _(second_barrier):
            for p in partners:
                pl.semaphore_signal(
                    second_barrier, inc=1,
                    device_id=(p,), device_id_type=pl.DeviceIdType.MESH,
                )
            pl.semaphore_wait(second_barrier, N_ROUNDS)

    return pl.pallas_call(
        body,
        out_shape=jax.ShapeDtypeStruct((n, h), jnp.float32),
        in_specs=[
            pl.BlockSpec(memory_space=pltpu.VMEM),
            pl.BlockSpec(memory_space=pltpu.VMEM),
            pl.BlockSpec(memory_space=pltpu.VMEM),
        ],
        out_specs=pl.BlockSpec(memory_space=pltpu.VMEM),
        scratch_shapes=[
            pltpu.VMEM((N_ROUNDS, n, h), jnp.float32),
            pltpu.SemaphoreType.DMA((N_ROUNDS,)),
            pltpu.SemaphoreType.DMA((N_ROUNDS,)),
        ],
        compiler_params=pltpu.CompilerParams(collective_id=0),
    )(x, route_idx, expert_W)


# device time: 24736 ns/iter; 3.2123x vs baseline; 1.2274x over previous
import functools

import jax
import jax.numpy as jnp
from jax import lax
from jax.experimental import pallas as pl
from jax.experimental.pallas import tpu as pltpu

N_DEV = 16
SCHED = (1, 3, 4, 8)
N_ROUNDS = len(SCHED)


def kernel(x, router_W, route_idx, expert_W):
    n, d = x.shape
    e_per, _, h = expert_W.shape
    hh = h // 2

    def body(x_ref, ridx_ref, ew_ref, out_ref,
             recv_a, recv_b, send_sems_a, recv_sems_a, send_sems_b,
             recv_sems_b):
        my = lax.axis_index("i")
        partners = [jnp.bitwise_xor(my, m) for m in SCHED]

        barrier_sem = pltpu.get_barrier_semaphore()
        for p in partners:
            pl.semaphore_signal(
                barrier_sem, inc=1,
                device_id=(p,), device_id_type=pl.DeviceIdType.MESH,
            )
        pl.semaphore_wait(barrier_sem, N_ROUNDS)

        ridx = ridx_ref[:, :1]
        partial = jnp.zeros((n, h), jnp.float32)
        for j in range(e_per):
            e_id = my * e_per + j
            mask = (ridx == e_id).astype(jnp.float32)
            xm = x_ref[:, :] * mask
            partial = partial + jnp.dot(
                xm, ew_ref[j], preferred_element_type=jnp.float32
            )
        out_ref[:, :] = partial

        for r in range(N_ROUNDS):
            rd_a = pltpu.make_async_remote_copy(
                src_ref=out_ref.at[:, pl.ds(0, hh)],
                dst_ref=recv_a.at[r],
                send_sem=send_sems_a.at[r],
                recv_sem=recv_sems_a.at[r],
                device_id=(partners[r],),
                device_id_type=pl.DeviceIdType.MESH,
            )
            rd_b = pltpu.make_async_remote_copy(
                src_ref=out_ref.at[:, pl.ds(hh, hh)],
                dst_ref=recv_b.at[r],
                send_sem=send_sems_b.at[r],
                recv_sem=recv_sems_b.at[r],
                device_id=(partners[N_ROUNDS - 1 - r],),
                device_id_type=pl.DeviceIdType.MESH,
            )
            rd_a.start()
            rd_b.start()
            rd_a.wait()
            out_ref[:, pl.ds(0, hh)] += recv_a[r]
            rd_b.wait()
            out_ref[:, pl.ds(hh, hh)] += recv_b[r]

        @functools.partial(
            pl.run_scoped, second_barrier=pltpu.SemaphoreType.REGULAR
        )
        def _(second_barrier):
            for p in partners:
                pl.semaphore_signal(
                    second_barrier, inc=1,
                    device_id=(p,), device_id_type=pl.DeviceIdType.MESH,
                )
            pl.semaphore_wait(second_barrier, N_ROUNDS)

    return pl.pallas_call(
        body,
        out_shape=jax.ShapeDtypeStruct((n, h), jnp.float32),
        in_specs=[
            pl.BlockSpec(memory_space=pltpu.VMEM),
            pl.BlockSpec(memory_space=pltpu.VMEM),
            pl.BlockSpec(memory_space=pltpu.VMEM),
        ],
        out_specs=pl.BlockSpec(memory_space=pltpu.VMEM),
        scratch_shapes=[
            pltpu.VMEM((N_ROUNDS, n, hh), jnp.float32),
            pltpu.VMEM((N_ROUNDS, n, hh), jnp.float32),
            pltpu.SemaphoreType.DMA((N_ROUNDS,)),
            pltpu.SemaphoreType.DMA((N_ROUNDS,)),
            pltpu.SemaphoreType.DMA((N_ROUNDS,)),
            pltpu.SemaphoreType.DMA((N_ROUNDS,)),
        ],
        compiler_params=pltpu.CompilerParams(collective_id=0),
    )(x, route_idx, expert_W)


# device time: 23205 ns/iter; 3.4243x vs baseline; 1.0660x over previous
import functools

import jax
import jax.numpy as jnp
from jax import lax
from jax.experimental import pallas as pl
from jax.experimental.pallas import tpu as pltpu

N_DEV = 16
SCHED = (1, 3, 4, 8)
N_ROUNDS = len(SCHED)


def kernel(x, router_W, route_idx, expert_W):
    n, d = x.shape
    e_per, _, h = expert_W.shape
    hh = h // 2

    def body(x_ref, ridx_ref, ew_ref, out_ref,
             recv_a, recv_b, send_sems_a, recv_sems_a, send_sems_b,
             recv_sems_b):
        my = lax.axis_index("i")
        partners = [jnp.bitwise_xor(my, m) for m in SCHED]

        barrier_sem = pltpu.get_barrier_semaphore()
        for p in partners:
            pl.semaphore_signal(
                barrier_sem, inc=1,
                device_id=(p,), device_id_type=pl.DeviceIdType.MESH,
            )

        ridx = ridx_ref[:, :1]
        partial = jnp.zeros((n, h), jnp.float32)
        for j in range(e_per):
            e_id = my * e_per + j
            mask = (ridx == e_id).astype(jnp.float32)
            xm = x_ref[:, :] * mask
            partial = partial + jnp.dot(
                xm, ew_ref[j], preferred_element_type=jnp.float32
            )
        out_ref[:, :] = partial
        pl.semaphore_wait(barrier_sem, N_ROUNDS)

        for r in range(N_ROUNDS):
            rd_a = pltpu.make_async_remote_copy(
                src_ref=out_ref.at[:, pl.ds(0, hh)],
                dst_ref=recv_a.at[r],
                send_sem=send_sems_a.at[r],
                recv_sem=recv_sems_a.at[r],
                device_id=(partners[r],),
                device_id_type=pl.DeviceIdType.MESH,
            )
            rd_b = pltpu.make_async_remote_copy(
                src_ref=out_ref.at[:, pl.ds(hh, hh)],
                dst_ref=recv_b.at[r],
                send_sem=send_sems_b.at[r],
                recv_sem=recv_sems_b.at[r],
                device_id=(partners[N_ROUNDS - 1 - r],),
                device_id_type=pl.DeviceIdType.MESH,
            )
            rd_a.start()
            rd_b.start()
            rd_a.wait()
            out_ref[:, pl.ds(0, hh)] += recv_a[r]
            rd_b.wait()
            out_ref[:, pl.ds(hh, hh)] += recv_b[r]

    return pl.pallas_call(
        body,
        out_shape=jax.ShapeDtypeStruct((n, h), jnp.float32),
        in_specs=[
            pl.BlockSpec(memory_space=pltpu.VMEM),
            pl.BlockSpec(memory_space=pltpu.VMEM),
            pl.BlockSpec(memory_space=pltpu.VMEM),
        ],
        out_specs=pl.BlockSpec(memory_space=pltpu.VMEM),
        scratch_shapes=[
            pltpu.VMEM((N_ROUNDS, n, hh), jnp.float32),
            pltpu.VMEM((N_ROUNDS, n, hh), jnp.float32),
            pltpu.SemaphoreType.DMA((N_ROUNDS,)),
            pltpu.SemaphoreType.DMA((N_ROUNDS,)),
            pltpu.SemaphoreType.DMA((N_ROUNDS,)),
            pltpu.SemaphoreType.DMA((N_ROUNDS,)),
        ],
        compiler_params=pltpu.CompilerParams(collective_id=0),
    )(x, route_idx, expert_W)
